# baseline (speedup 1.0000x reference)
import functools

import jax
import jax.numpy as jnp
from jax import lax
from jax.experimental import pallas as pl
from jax.experimental.pallas import tpu as pltpu
from jax.experimental.pallas import tpu_sc as plsc

_HIGHEST = lax.Precision.HIGHEST


def _make_sc_mask_kernel(BS, H, W, RH, RW):
    info = plsc.get_sparse_core_info()
    NC, NS = info.num_cores, info.num_subcores
    assert NC * NS == BS, (NC, NS, BS)
    N = RH * RW
    sh, sw = H // RH, W // RW
    n_chunks = N // 16
    n_dma = n_chunks // 8

    mesh = plsc.VectorSubcoreMesh(core_axis_name="c", subcore_axis_name="s")

    @functools.partial(
        pl.kernel,
        mesh=mesh,
        out_type=[
            jax.ShapeDtypeStruct((BS * N,), jnp.float32),
            jax.ShapeDtypeStruct((BS, 16), jnp.float32),
        ],
        scratch_types=[
            pltpu.VMEM((n_dma, 128), jnp.int32),
            pltpu.VMEM((N,), jnp.float32),
            pltpu.VMEM((N,), jnp.float32),
            pltpu.VMEM((16,), jnp.float32),
            pltpu.SemaphoreType.DMA,
        ],
    )
    def sc_mask(sy_flat, mask_out, cnt_out, idx_v, vals_v, mask_v, cnt_v, sem):
        wid = lax.axis_index("s") * NC + lax.axis_index("c")
        base = wid * (H * W)
        lane = lax.broadcasted_iota(jnp.int32, (16,), 0)
        for c in range(n_chunks):
            i = (16 * c) // RW
            j0 = (16 * c) % RW
            e0 = sh * i * W + sw * j0
            idx_v[c // 8, pl.ds((c % 8) * 16, 16)] = base + e0 + sw * lane
        copies = [
            pltpu.async_copy(
                sy_flat.at[idx_v.at[d]], vals_v.at[pl.ds(d * 128, 128)], sem)
            for d in range(n_dma)
        ]
        for cp in copies:
            cp.wait()
        acc = jnp.zeros((16,), jnp.float32)
        for c in range(n_chunks):
            vals = vals_v[pl.ds(c * 16, 16)]
            m = jnp.where(vals == 1.0, 1.0, 0.0).astype(jnp.float32)
            mask_v[pl.ds(c * 16, 16)] = m
            acc = acc + m
        cnt_v[...] = acc
        pltpu.sync_copy(mask_v, mask_out.at[pl.ds(wid * N, N)])
        pltpu.sync_copy(cnt_v, cnt_out.at[wid])

    return sc_mask


def _compute_mask(s_y, BS, H, W, RH, RW):
    sy_flat = s_y.reshape(-1)
    return _make_sc_mask_kernel(BS, H, W, RH, RW)(sy_flat)


def _stats_body(m_ref, f_ref, o_ref):
    s = pl.program_id(1)
    part = lax.dot_general(
        m_ref[0], f_ref[0], (((1,), (0,)), ((), ())),
        precision=_HIGHEST, preferred_element_type=jnp.float32)

    @pl.when(s == 0)
    def _init():
        o_ref[0] = part

    @pl.when(s != 0)
    def _acc():
        o_ref[0] += part


def _masked_sum(mask3, s_f, B, S, N, D):
    return pl.pallas_call(
        _stats_body,
        grid=(B, S),
        in_specs=[
            pl.BlockSpec((1, 1, N), lambda b, s: (b * S + s, 0, 0)),
            pl.BlockSpec((1, N, D), lambda b, s: (b * S + s, 0, 0)),
        ],
        out_specs=pl.BlockSpec((1, 1, D), lambda b, s: (b, 0, 0)),
        out_shape=jax.ShapeDtypeStruct((B, 1, D), jnp.float32),
    )(mask3, s_f)


def _select_body(spsum_ref, cnt_ref, proto_ref, o_ref, *, K, D):
    spsum = spsum_ref[...]
    cnt = cnt_ref[...]
    p = proto_ref[...]
    num_fore = jnp.sum(cnt, axis=1, keepdims=True)
    sp = spsum / (num_fore + 1e-4)
    colnorm = jnp.sqrt(jnp.sum(p * p, axis=0, keepdims=True))
    bank = p / jnp.maximum(colnorm, 1e-12)
    spn = sp / jnp.maximum(
        jnp.sqrt(jnp.sum(sp * sp, axis=1, keepdims=True)), 1e-12)
    sim = lax.dot_general(
        spn, bank, (((1,), (0,)), ((), ())),
        precision=_HIGHEST, preferred_element_type=jnp.float32)
    maxv = jnp.max(sim, axis=1, keepdims=True)
    iota = lax.broadcasted_iota(jnp.int32, sim.shape, 1)
    idx = jnp.min(jnp.where(sim == maxv, iota, K), axis=1, keepdims=True)
    onehot = (iota == idx).astype(jnp.float32)
    new_sp = lax.dot_general(
        onehot, bank, (((1,), (1,)), ((), ())),
        precision=_HIGHEST, preferred_element_type=jnp.float32)
    nsp = new_sp / jnp.maximum(
        jnp.sqrt(jnp.sum(new_sp * new_sp, axis=1, keepdims=True)), 1e-12)
    sign = (num_fore > 0.5).astype(jnp.float32)
    o_ref[:, 0, :] = nsp * sign * (float(D) ** 0.5)


def _bank_select(spsum, cnt, prototype, B, K, D):
    return pl.pallas_call(
        functools.partial(_select_body, K=K, D=D),
        out_shape=jax.ShapeDtypeStruct((B, 1, D), jnp.float32),
    )(spsum, cnt, prototype)


def _fused_body(sf_ref, x_ref, nsp_ref, wd_ref, bd_ref, wu_ref, bu_ref,
                o_ref, *, BS):
    g = pl.program_id(0)
    f = jnp.where(g < BS, sf_ref[0], x_ref[0])
    nsp = nsp_ref[0]
    sq = jnp.sum(f * f, axis=1, keepdims=True)
    inv = 1.0 / jnp.maximum(jnp.sqrt(sq), 1e-12)
    dot = jnp.sum(f * nsp, axis=1, keepdims=True)
    sm = jnp.clip(dot * inv, 0.0, 6.0)
    fe = f * (1.0 + sm)
    h = lax.dot_general(
        fe, wd_ref[...], (((1,), (1,)), ((), ())),
        precision=_HIGHEST, preferred_element_type=jnp.float32)
    h = jnp.maximum(h + bd_ref[...], 0.0)
    o = lax.dot_general(
        h, wu_ref[...], (((1,), (1,)), ((), ())),
        precision=_HIGHEST, preferred_element_type=jnp.float32)
    o_ref[0] = o + bu_ref[...]


def _fused_enhance_mlp(s_f, x, nsp_eff, W_down, b_down2, W_up, b_up2,
                       B, S, N, D, HID, T):
    BS = B * S
    NT = N // T
    G = BS + B

    def sf_map(g, t):
        return (jnp.minimum(g, BS - 1), jnp.where(g < BS, t, NT - 1), 0)

    def x_map(g, t):
        return (jnp.maximum(g - BS, 0), jnp.where(g < BS, 0, t), 0)

    def nsp_map(g, t):
        return (jnp.where(g < BS, g // S, g - BS), 0, 0)

    return pl.pallas_call(
        functools.partial(_fused_body, BS=BS),
        grid=(G, NT),
        in_specs=[
            pl.BlockSpec((1, T, D), sf_map),
            pl.BlockSpec((1, T, D), x_map),
            pl.BlockSpec((1, 1, D), nsp_map),
            pl.BlockSpec((HID, D), lambda g, t: (0, 0)),
            pl.BlockSpec((1, HID), lambda g, t: (0, 0)),
            pl.BlockSpec((D, HID), lambda g, t: (0, 0)),
            pl.BlockSpec((1, D), lambda g, t: (0, 0)),
        ],
        out_specs=pl.BlockSpec((1, T, D), lambda g, t: (g, t, 0)),
        out_shape=jax.ShapeDtypeStruct((G, N, D), jnp.float32),
    )(s_f, x, nsp_eff, W_down, b_down2, W_up, b_up2)


def kernel(x, s_f, s_y, prototype, W_down, b_down, W_up, b_up):
    B, N, D = x.shape
    BS = s_f.shape[0]
    S = BS // B
    _, _, H, W = s_y.shape
    K = prototype.shape[1]
    HID = W_down.shape[0]
    RH = RW = int(round(N ** 0.5))

    mask_flat, cnt = _compute_mask(s_y, BS, H, W, RH, RW)
    mask3 = mask_flat.reshape(BS, 1, N)
    spsum = _masked_sum(mask3, s_f, B, S, N, D)
    nsp_eff = _bank_select(spsum.reshape(B, D), cnt.reshape(B, S * 16),
                           prototype, B, K, D)
    out = _fused_enhance_mlp(
        s_f, x, nsp_eff, W_down, b_down.reshape(1, HID), W_up,
        b_up.reshape(1, D), B, S, N, D, HID, T=512)
    return out


# speedup vs baseline: 2.3829x; 2.3829x over previous
import functools

import jax
import jax.numpy as jnp
from jax import lax
from jax.experimental import pallas as pl
from jax.experimental.pallas import tpu as pltpu
from jax.experimental.pallas import tpu_sc as plsc

_HIGHEST = lax.Precision.HIGHEST
_DEFAULT = lax.Precision.DEFAULT


def _make_sc_mask_kernel(BS, H, W, RH, RW):
    info = plsc.get_sparse_core_info()
    NC, NS = info.num_cores, info.num_subcores
    assert NC * NS == BS, (NC, NS, BS)
    N = RH * RW
    sh, sw = H // RH, W // RW
    n_chunks = N // 16
    n_dma = n_chunks // 8

    mesh = plsc.VectorSubcoreMesh(core_axis_name="c", subcore_axis_name="s")

    @functools.partial(
        pl.kernel,
        mesh=mesh,
        out_type=[
            jax.ShapeDtypeStruct((BS * N,), jnp.float32),
            jax.ShapeDtypeStruct((BS, 16), jnp.float32),
        ],
        scratch_types=[
            pltpu.VMEM((n_dma, 128), jnp.int32),
            pltpu.VMEM((N,), jnp.float32),
            pltpu.VMEM((N,), jnp.float32),
            pltpu.VMEM((16,), jnp.float32),
            pltpu.SemaphoreType.DMA,
        ],
    )
    def sc_mask(sy_flat, mask_out, cnt_out, idx_v, vals_v, mask_v, cnt_v, sem):
        wid = lax.axis_index("s") * NC + lax.axis_index("c")
        base = wid * (H * W)
        lane = lax.broadcasted_iota(jnp.int32, (16,), 0)
        for c in range(n_chunks):
            i = (16 * c) // RW
            j0 = (16 * c) % RW
            e0 = sh * i * W + sw * j0
            idx_v[c // 8, pl.ds((c % 8) * 16, 16)] = base + e0 + sw * lane
        copies = [
            pltpu.async_copy(
                sy_flat.at[idx_v.at[d]], vals_v.at[pl.ds(d * 128, 128)], sem)
            for d in range(n_dma)
        ]
        for cp in copies:
            cp.wait()
        acc = jnp.zeros((16,), jnp.float32)
        for c in range(n_chunks):
            vals = vals_v[pl.ds(c * 16, 16)]
            m = jnp.where(vals == 1.0, 1.0, 0.0).astype(jnp.float32)
            mask_v[pl.ds(c * 16, 16)] = m
            acc = acc + m
        cnt_v[...] = acc
        pltpu.sync_copy(mask_v, mask_out.at[pl.ds(wid * N, N)])
        pltpu.sync_copy(cnt_v, cnt_out.at[wid])

    return sc_mask


def _compute_mask(s_y, BS, H, W, RH, RW):
    sy_flat = s_y.reshape(-1)
    return _make_sc_mask_kernel(BS, H, W, RH, RW)(sy_flat)


def _stats_body(m_ref, f_ref, o_ref):
    s = pl.program_id(1)
    part = lax.dot_general(
        m_ref[0], f_ref[0], (((1,), (0,)), ((), ())),
        precision=_DEFAULT, preferred_element_type=jnp.float32)

    @pl.when(s == 0)
    def _init():
        o_ref[0] = part

    @pl.when(s != 0)
    def _acc():
        o_ref[0] += part


def _masked_sum(mask3, s_f, B, S, N, D):
    return pl.pallas_call(
        _stats_body,
        grid=(B, S),
        in_specs=[
            pl.BlockSpec((1, 1, N), lambda b, s: (b * S + s, 0, 0)),
            pl.BlockSpec((1, N, D), lambda b, s: (b * S + s, 0, 0)),
        ],
        out_specs=pl.BlockSpec((1, 1, D), lambda b, s: (b, 0, 0)),
        out_shape=jax.ShapeDtypeStruct((B, 1, D), jnp.float32),
    )(mask3, s_f)


def _select_body(spsum_ref, cnt_ref, proto_ref, o_ref, *, K, D):
    spsum = spsum_ref[...]
    cnt = cnt_ref[...]
    p = proto_ref[...]
    num_fore = jnp.sum(cnt, axis=1, keepdims=True)
    sp = spsum / (num_fore + 1e-4)
    colnorm = jnp.sqrt(jnp.sum(p * p, axis=0, keepdims=True))
    bank = p / jnp.maximum(colnorm, 1e-12)
    spn = sp / jnp.maximum(
        jnp.sqrt(jnp.sum(sp * sp, axis=1, keepdims=True)), 1e-12)
    sim = lax.dot_general(
        spn, bank, (((1,), (0,)), ((), ())),
        precision=_HIGHEST, preferred_element_type=jnp.float32)
    maxv = jnp.max(sim, axis=1, keepdims=True)
    iota = lax.broadcasted_iota(jnp.int32, sim.shape, 1)
    idx = jnp.min(jnp.where(sim == maxv, iota, K), axis=1, keepdims=True)
    onehot = (iota == idx).astype(jnp.float32)
    new_sp = lax.dot_general(
        onehot, bank, (((1,), (1,)), ((), ())),
        precision=_HIGHEST, preferred_element_type=jnp.float32)
    nsp = new_sp / jnp.maximum(
        jnp.sqrt(jnp.sum(new_sp * new_sp, axis=1, keepdims=True)), 1e-12)
    sign = (num_fore > 0.5).astype(jnp.float32)
    o_ref[:, 0, :] = nsp * sign * (float(D) ** 0.5)


def _bank_select(spsum, cnt, prototype, B, K, D):
    return pl.pallas_call(
        functools.partial(_select_body, K=K, D=D),
        out_shape=jax.ShapeDtypeStruct((B, 1, D), jnp.float32),
    )(spsum, cnt, prototype)


def _fused_body(sf_ref, x_ref, nsp_ref, wd_ref, bd_ref, wu_ref, bu_ref,
                o_ref, *, BS):
    g = pl.program_id(0)
    f = jnp.where(g < BS, sf_ref[0], x_ref[0])
    nsp = nsp_ref[0]
    sq = jnp.sum(f * f, axis=1, keepdims=True)
    inv = 1.0 / jnp.maximum(jnp.sqrt(sq), 1e-12)
    dot = jnp.sum(f * nsp, axis=1, keepdims=True)
    sm = jnp.clip(dot * inv, 0.0, 6.0)
    fe = f * (1.0 + sm)
    h = lax.dot_general(
        fe, wd_ref[...], (((1,), (1,)), ((), ())),
        precision=_DEFAULT, preferred_element_type=jnp.float32)
    h = jnp.maximum(h + bd_ref[...], 0.0)
    o = lax.dot_general(
        h, wu_ref[...], (((1,), (1,)), ((), ())),
        precision=_DEFAULT, preferred_element_type=jnp.float32)
    o_ref[0] = o + bu_ref[...]


def _fused_enhance_mlp(s_f, x, nsp_eff, W_down, b_down2, W_up, b_up2,
                       B, S, N, D, HID, T):
    BS = B * S
    NT = N // T
    G = BS + B

    def sf_map(g, t):
        return (jnp.minimum(g, BS - 1), jnp.where(g < BS, t, NT - 1), 0)

    def x_map(g, t):
        return (jnp.maximum(g - BS, 0), jnp.where(g < BS, 0, t), 0)

    def nsp_map(g, t):
        return (jnp.where(g < BS, g // S, g - BS), 0, 0)

    return pl.pallas_call(
        functools.partial(_fused_body, BS=BS),
        grid=(G, NT),
        in_specs=[
            pl.BlockSpec((1, T, D), sf_map),
            pl.BlockSpec((1, T, D), x_map),
            pl.BlockSpec((1, 1, D), nsp_map),
            pl.BlockSpec((HID, D), lambda g, t: (0, 0)),
            pl.BlockSpec((1, HID), lambda g, t: (0, 0)),
            pl.BlockSpec((D, HID), lambda g, t: (0, 0)),
            pl.BlockSpec((1, D), lambda g, t: (0, 0)),
        ],
        out_specs=pl.BlockSpec((1, T, D), lambda g, t: (g, t, 0)),
        out_shape=jax.ShapeDtypeStruct((G, N, D), jnp.float32),
    )(s_f, x, nsp_eff, W_down, b_down2, W_up, b_up2)


def kernel(x, s_f, s_y, prototype, W_down, b_down, W_up, b_up):
    B, N, D = x.shape
    BS = s_f.shape[0]
    S = BS // B
    _, _, H, W = s_y.shape
    K = prototype.shape[1]
    HID = W_down.shape[0]
    RH = RW = int(round(N ** 0.5))

    mask_flat, cnt = _compute_mask(s_y, BS, H, W, RH, RW)
    mask3 = mask_flat.reshape(BS, 1, N)
    spsum = _masked_sum(mask3, s_f, B, S, N, D)
    nsp_eff = _bank_select(spsum.reshape(B, D), cnt.reshape(B, S * 16),
                           prototype, B, K, D)
    out = _fused_enhance_mlp(
        s_f, x, nsp_eff, W_down, b_down.reshape(1, HID), W_up,
        b_up.reshape(1, D), B, S, N, D, HID, T=512)
    return out
